# Initial kernel scaffold; baseline (speedup 1.0000x reference)
import jax
import jax.numpy as jnp
from jax.experimental import pallas as pl


def kernel(prior_info, table):
    raise NotImplementedError("write your pallas kernel here")


# speedup vs baseline: 3.6136x; 3.6136x over previous
import functools

import jax
import jax.numpy as jnp
from jax import lax
from jax.experimental import pallas as pl
from jax.experimental.pallas import tpu as pltpu
from jax.experimental.pallas import tpu_sc as plsc


@functools.lru_cache(maxsize=None)
def _make_gather(B, D):
    info = plsc.get_sparse_core_info()
    NC, NS = info.num_cores, info.num_subcores
    NW = NC * NS
    assert B % NW == 0
    BPW = B // NW
    CH = 800
    assert BPW % CH == 0
    NCH = BPW // CH
    mesh = plsc.VectorSubcoreMesh(core_axis_name="c", subcore_axis_name="s")

    @functools.partial(
        pl.kernel,
        mesh=mesh,
        out_type=jax.ShapeDtypeStruct((B, D), jnp.float32),
        scratch_types=[
            pltpu.VMEM((BPW,), jnp.int32),
            pltpu.VMEM((CH, D), jnp.float32),
            pltpu.SemaphoreType.DMA,
        ],
        compiler_params=pltpu.CompilerParams(use_tc_tiling_on_sc=False),
    )
    def k(idx_hbm, table_hbm, out_hbm, idx_v, rows_v, sem):
        wid = lax.axis_index("s") * NC + lax.axis_index("c")
        base = wid * BPW
        pltpu.sync_copy(idx_hbm.at[pl.ds(base, BPW)], idx_v)

        def body(g, carry):
            off = pl.multiple_of(g * CH, 8)
            pltpu.async_copy(
                table_hbm.at[idx_v.at[pl.ds(off, CH)]], rows_v, sem
            ).wait()
            pltpu.sync_copy(rows_v, out_hbm.at[pl.ds(base + off, CH)])
            return carry

        lax.fori_loop(0, NCH, body, 0)

    return k


def kernel(prior_info, table):
    Bt, S = prior_info.shape
    V, D = table.shape
    idx = prior_info.reshape(-1).astype(jnp.int32)
    out = _make_gather(idx.shape[0], D)(idx, table)
    return out.reshape(Bt, S, D)


# speedup vs baseline: 4.3465x; 1.2028x over previous
import functools

import jax
import jax.numpy as jnp
from jax import lax
from jax.experimental import pallas as pl
from jax.experimental.pallas import tpu as pltpu
from jax.experimental.pallas import tpu_sc as plsc

_L = 16


@functools.lru_cache(maxsize=None)
def _make_lookup(Bt, S, V, D):
    info = plsc.get_sparse_core_info()
    NC, NS = info.num_cores, info.num_subcores
    NW = NC * NS
    assert Bt % (NW * 128) == 0 and D % 8 == 0
    BB = Bt // NW
    DT = D // 8
    mesh = plsc.VectorSubcoreMesh(core_axis_name="c", subcore_axis_name="s")

    @functools.partial(
        pl.kernel,
        mesh=mesh,
        out_type=jax.ShapeDtypeStruct((S, DT, NW, 8, 128), jnp.float32),
        scratch_types=[
            pltpu.VMEM((BB * S,), jnp.int32),
            pltpu.VMEM((V * D,), jnp.float32),
            pltpu.VMEM((DT, 8, 128), jnp.float32),
            pltpu.VMEM((DT, 8, 128), jnp.float32),
            pltpu.SemaphoreType.DMA,
            pltpu.SemaphoreType.DMA,
        ],
        compiler_params=pltpu.CompilerParams(
            use_tc_tiling_on_sc=False, needs_layout_passes=False
        ),
    )
    def k(idx_hbm, table_hbm, out_hbm, idx_v, tab_v, pl_a, pl_b, sem_a, sem_b):
        wid = lax.axis_index("s") * NC + lax.axis_index("c")
        pltpu.sync_copy(idx_hbm.at[pl.ds(wid * BB * S, BB * S)], idx_v)
        pltpu.sync_copy(table_hbm, tab_v)
        iota_s = lax.iota(jnp.int32, _L) * S

        def compute(s, plane):
            for bs in range(BB // _L):
                pos = iota_s + (bs * _L * S + s)
                iv = plsc.load_gather(idx_v, [pos])
                ivd = iv * D
                for d in range(D):
                    v = plsc.load_gather(tab_v, [ivd + d])
                    plane[d // 8, d % 8, pl.ds(bs * _L, _L)] = v

        def fire(s, plane, sem):
            for t in range(DT):
                pltpu.async_copy(plane.at[t], out_hbm.at[s, t, wid], sem)

        def drain(plane, sem):
            for t in range(DT):
                pltpu.make_async_copy(plane.at[t], out_hbm.at[0, t, wid], sem).wait()

        compute(0, pl_a)
        fire(0, pl_a, sem_a)
        compute(1, pl_b)
        fire(1, pl_b, sem_b)

        def outer(i, c):
            s = 2 + i * 2
            drain(pl_a, sem_a)
            compute(s, pl_a)
            fire(s, pl_a, sem_a)
            drain(pl_b, sem_b)
            compute(s + 1, pl_b)
            fire(s + 1, pl_b, sem_b)
            return c

        lax.fori_loop(0, (S - 2) // 2, outer, 0)
        drain(pl_a, sem_a)
        drain(pl_b, sem_b)

    return k


def kernel(prior_info, table):
    Bt, S = prior_info.shape
    V, D = table.shape
    idx = prior_info.reshape(-1).astype(jnp.int32)
    out5 = _make_lookup(Bt, S, V, D)(idx, table.reshape(-1))
    return out5.transpose(2, 4, 0, 1, 3).reshape(Bt, S, D)


# speedup vs baseline: 5.4510x; 1.2541x over previous
import functools

import jax
import jax.numpy as jnp
from jax import lax
from jax.experimental import pallas as pl
from jax.experimental.pallas import tpu as pltpu
from jax.experimental.pallas import tpu_sc as plsc

_L = 16


@functools.lru_cache(maxsize=None)
def _make_lookup(Bt, S, V, D):
    info = plsc.get_sparse_core_info()
    NC, NS = info.num_cores, info.num_subcores
    NW = NC * NS
    assert Bt % (NW * 128) == 0 and D % 8 == 0
    BB = Bt // NW
    DT = D // 8
    mesh = plsc.VectorSubcoreMesh(core_axis_name="c", subcore_axis_name="s")

    @functools.partial(
        pl.kernel,
        mesh=mesh,
        out_type=jax.ShapeDtypeStruct((S, DT, NW, 8, 128), jnp.float32),
        scratch_types=[
            pltpu.VMEM((BB * S,), jnp.int32),
            pltpu.VMEM((V * D,), jnp.float32),
            pltpu.VMEM((DT, 8, 128), jnp.float32),
            pltpu.VMEM((DT, 8, 128), jnp.float32),
            pltpu.SemaphoreType.DMA,
            pltpu.SemaphoreType.DMA,
        ],
        compiler_params=pltpu.CompilerParams(
            use_tc_tiling_on_sc=False, needs_layout_passes=False
        ),
    )
    def k(idx_hbm, table_hbm, out_hbm, idx_v, tab_v, pl_a, pl_b, sem_a, sem_b):
        wid = lax.axis_index("s") * NC + lax.axis_index("c")
        pltpu.sync_copy(idx_hbm.at[pl.ds(wid * BB * S, BB * S)], idx_v)
        pltpu.sync_copy(table_hbm, tab_v)
        iota_s = lax.iota(jnp.int32, _L) * S

        NG = BB // _L

        def compute(s, plane):
            ivds = []
            for bs in range(NG):
                pos = iota_s + (bs * _L * S + s)
                iv = plsc.load_gather(idx_v, [pos])
                ivds.append(iv * D)
            for d in range(D):
                vs = [plsc.load_gather(tab_v, [ivd + d]) for ivd in ivds]
                for bs in range(NG):
                    plane[d // 8, d % 8, pl.ds(bs * _L, _L)] = vs[bs]

        def fire(s, plane, sem):
            for t in range(DT):
                pltpu.async_copy(plane.at[t], out_hbm.at[s, t, wid], sem)

        def drain(plane, sem):
            for t in range(DT):
                pltpu.make_async_copy(plane.at[t], out_hbm.at[0, t, wid], sem).wait()

        compute(0, pl_a)
        fire(0, pl_a, sem_a)
        compute(1, pl_b)
        fire(1, pl_b, sem_b)

        def outer(i, c):
            s = 2 + i * 2
            drain(pl_a, sem_a)
            compute(s, pl_a)
            fire(s, pl_a, sem_a)
            drain(pl_b, sem_b)
            compute(s + 1, pl_b)
            fire(s + 1, pl_b, sem_b)
            return c

        lax.fori_loop(0, (S - 2) // 2, outer, 0)
        drain(pl_a, sem_a)
        drain(pl_b, sem_b)

    return k


def kernel(prior_info, table):
    Bt, S = prior_info.shape
    V, D = table.shape
    idx = prior_info.reshape(-1).astype(jnp.int32)
    out5 = _make_lookup(Bt, S, V, D)(idx, table.reshape(-1))
    return out5.transpose(2, 4, 0, 1, 3).reshape(Bt, S, D)


# speedup vs baseline: 27.4641x; 5.0384x over previous
import functools

import jax
import jax.numpy as jnp
from jax import lax
from jax.experimental import pallas as pl
from jax.experimental.pallas import tpu as pltpu
from jax.experimental.pallas import tpu_sc as plsc

_L = 16


@functools.lru_cache(maxsize=None)
def _make_lookup(Bt, S, V, D):
    info = plsc.get_sparse_core_info()
    NC, NS = info.num_cores, info.num_subcores
    NW = NC * NS
    assert Bt % (NW * 128) == 0 and D % 8 == 0
    BB = Bt // NW
    DT = D // 8
    DP = D + 1
    mesh = plsc.VectorSubcoreMesh(core_axis_name="c", subcore_axis_name="s")

    @functools.partial(
        pl.kernel,
        mesh=mesh,
        out_type=jax.ShapeDtypeStruct((S, DT, NW, 8, 128), jnp.float32),
        scratch_types=[
            pltpu.VMEM((BB * S,), jnp.int32),
            pltpu.VMEM((V * DP,), jnp.float32),
            pltpu.VMEM((DT, 8, 128), jnp.float32),
            pltpu.VMEM((DT, 8, 128), jnp.float32),
            pltpu.SemaphoreType.DMA,
            pltpu.SemaphoreType.DMA,
        ],
        compiler_params=pltpu.CompilerParams(
            use_tc_tiling_on_sc=False, needs_layout_passes=False
        ),
    )
    def k(idx_hbm, table_hbm, out_hbm, idx_v, tab_v, pl_a, pl_b, sem_a, sem_b):
        wid = lax.axis_index("s") * NC + lax.axis_index("c")
        pltpu.sync_copy(idx_hbm.at[pl.ds(wid * BB * S, BB * S)], idx_v)
        pltpu.sync_copy(table_hbm, tab_v)
        iota_s = lax.iota(jnp.int32, _L) * S

        NG = BB // _L

        def compute(s, plane):
            ivds = []
            for bs in range(NG):
                pos = iota_s + (bs * _L * S + s)
                iv = plsc.load_gather(idx_v, [pos])
                ivds.append(iv * DP)
            for d in range(D):
                vs = [plsc.load_gather(tab_v, [ivd + d]) for ivd in ivds]
                for bs in range(NG):
                    plane[d // 8, d % 8, pl.ds(bs * _L, _L)] = vs[bs]

        def fire(s, plane, sem):
            for t in range(DT):
                pltpu.async_copy(plane.at[t], out_hbm.at[s, t, wid], sem)

        def drain(plane, sem):
            for t in range(DT):
                pltpu.make_async_copy(plane.at[t], out_hbm.at[0, t, wid], sem).wait()

        compute(0, pl_a)
        fire(0, pl_a, sem_a)
        compute(1, pl_b)
        fire(1, pl_b, sem_b)

        def outer(i, c):
            s = 2 + i * 2
            drain(pl_a, sem_a)
            compute(s, pl_a)
            fire(s, pl_a, sem_a)
            drain(pl_b, sem_b)
            compute(s + 1, pl_b)
            fire(s + 1, pl_b, sem_b)
            return c

        lax.fori_loop(0, (S - 2) // 2, outer, 0)
        drain(pl_a, sem_a)
        drain(pl_b, sem_b)

    return k


def kernel(prior_info, table):
    Bt, S = prior_info.shape
    V, D = table.shape
    idx = prior_info.reshape(-1).astype(jnp.int32)
    table_padded = jnp.pad(table, ((0, 0), (0, 1))).reshape(-1)
    out5 = _make_lookup(Bt, S, V, D)(idx, table_padded)
    return out5.transpose(2, 4, 0, 1, 3).reshape(Bt, S, D)
